# baseline (speedup 1.0000x reference)
import functools

import jax
import jax.numpy as jnp
from jax.experimental import pallas as pl
from jax.experimental.pallas import tpu as pltpu

_NBUF = 4


def _gcn_kernel(al_hbm, ah_hbm, x_ref, wl_ref, wh_ref, wm_ref,
                avl_ref, avh_ref, avm_ref, a3_ref, out_ref,
                buf, ol_sc, xwl_ref, xwh_ref, om_ref, sems, *, g, bm):
    i = pl.program_id(0)
    ni = pl.num_programs(0)

    def _start(step, slot):
        @pl.when(step < g)
        def _():
            pltpu.make_async_copy(al_hbm.at[pl.ds(step * bm, bm), :],
                                  buf.at[slot], sems.at[slot]).start()

        @pl.when(step >= g)
        def _():
            pltpu.make_async_copy(ah_hbm.at[pl.ds((step - g) * bm, bm), :],
                                  buf.at[slot], sems.at[slot]).start()

    @pl.when(i == 0)
    def _():
        for j in range(_NBUF - 1):
            _start(j, j)

    @pl.when(i + _NBUF - 1 < ni)
    def _():
        step = i + _NBUF - 1
        _start(step, step % _NBUF)

    @pl.when(i == 0)
    def _():
        x = x_ref[...]
        xwl_ref[...] = jnp.dot(x, wl_ref[...],
                               preferred_element_type=jnp.float32).astype(jnp.bfloat16)
        xwh_ref[...] = jnp.dot(x, wh_ref[...],
                               preferred_element_type=jnp.float32).astype(jnp.bfloat16)
        om_ref[...] = jnp.maximum(
            jnp.dot(x, wm_ref[...], preferred_element_type=jnp.float32), 0.0)

    slot = i % _NBUF
    pltpu.make_async_copy(al_hbm.at[pl.ds(0, bm), :],
                          buf.at[slot], sems.at[slot]).wait()

    @pl.when(i < g)
    def _():
        a = buf[slot].astype(jnp.bfloat16)
        ol_sc[pl.ds(i * bm, bm), :] = jnp.maximum(
            jnp.dot(a, xwl_ref[...], preferred_element_type=jnp.float32), 0.0)

    @pl.when(i >= g)
    def _():
        j = i - g
        a = buf[slot].astype(jnp.bfloat16)
        oh = jnp.maximum(
            jnp.dot(a, xwh_ref[...], preferred_element_type=jnp.float32), 0.0)
        ol = ol_sc[pl.ds(j * bm, bm), :]
        om = om_ref[pl.ds(j * bm, bm), :]
        fl = jax.nn.sigmoid(jnp.sum(ol * avl_ref[...], axis=1, keepdims=True))
        fh = jax.nn.sigmoid(jnp.sum(oh * avh_ref[...], axis=1, keepdims=True))
        fm = jax.nn.sigmoid(jnp.sum(om * avm_ref[...], axis=1, keepdims=True))
        a3 = a3_ref[...]
        inv_t = 1.0 / 3.0
        l0 = (fl * a3[0, 0] + fh * a3[1, 0] + fm * a3[2, 0]) * inv_t
        l1 = (fl * a3[0, 1] + fh * a3[1, 1] + fm * a3[2, 1]) * inv_t
        l2 = (fl * a3[0, 2] + fh * a3[1, 2] + fm * a3[2, 2]) * inv_t
        m = jnp.maximum(jnp.maximum(l0, l1), l2)
        e0 = jnp.exp(l0 - m)
        e1 = jnp.exp(l1 - m)
        e2 = jnp.exp(l2 - m)
        scale = 3.0 / (e0 + e1 + e2)
        out_ref[...] = scale * (e0 * ol + e1 * oh + e2 * om)


def kernel(input, adj_low, adj_high, adj_low_unnormalized, W_low, W_high, W_mlp,
           att_vec_low, att_vec_high, att_vec_mlp, att_vec_3):
    n, din = input.shape
    dout = W_low.shape[1]

    bm = 200 if n % 200 == 0 else n
    g = n // bm

    avl = att_vec_low.reshape(1, dout)
    avh = att_vec_high.reshape(1, dout)
    avm = att_vec_mlp.reshape(1, dout)

    hbm = pl.BlockSpec(memory_space=pl.ANY)
    resident = lambda shape: pl.BlockSpec(shape, lambda i: (0, 0))
    row_out = pl.BlockSpec((bm, dout), lambda i: (jnp.maximum(i - g, 0), 0))

    out = pl.pallas_call(
        functools.partial(_gcn_kernel, g=g, bm=bm),
        grid=(2 * g,),
        in_specs=[
            hbm, hbm,
            resident((n, din)), resident((din, dout)), resident((din, dout)),
            resident((din, dout)),
            resident((1, dout)), resident((1, dout)), resident((1, dout)),
            resident((3, 3)),
        ],
        out_specs=row_out,
        out_shape=jax.ShapeDtypeStruct((n, dout), jnp.float32),
        scratch_shapes=[
            pltpu.VMEM((_NBUF, bm, n), jnp.float32),
            pltpu.VMEM((n, dout), jnp.float32),
            pltpu.VMEM((n, dout), jnp.bfloat16),
            pltpu.VMEM((n, dout), jnp.bfloat16),
            pltpu.VMEM((n, dout), jnp.float32),
            pltpu.SemaphoreType.DMA((_NBUF,)),
        ],
    )(adj_low, adj_high, input, W_low, W_high, W_mlp, avl, avh, avm, att_vec_3)
    return out


# speedup vs baseline: 1.0123x; 1.0123x over previous
import jax
import jax.numpy as jnp
from jax.experimental import pallas as pl
from jax.experimental.pallas import tpu as pltpu

_NBUF = 5


def _gcn_kernel(al_hbm, ah_hbm, x_ref, wl_ref, wh_ref, wm_ref,
                avl_ref, avh_ref, avm_ref, a3_ref, out_ref,
                al_buf, ah_buf, xwl_ref, xwh_ref, om_ref, sems):
    i = pl.program_id(0)
    ni = pl.num_programs(0)
    bm = out_ref.shape[0]

    def _start(step, slot):
        pltpu.make_async_copy(al_hbm.at[pl.ds(step * bm, bm), :],
                              al_buf.at[slot], sems.at[0, slot]).start()
        pltpu.make_async_copy(ah_hbm.at[pl.ds(step * bm, bm), :],
                              ah_buf.at[slot], sems.at[1, slot]).start()

    @pl.when(i == 0)
    def _():
        for j in range(_NBUF - 1):
            _start(j, j)

    @pl.when(i + _NBUF - 1 < ni)
    def _():
        step = i + _NBUF - 1
        _start(step, step % _NBUF)

    @pl.when(i == 0)
    def _():
        x = x_ref[...]
        xwl_ref[...] = jnp.dot(x, wl_ref[...],
                               preferred_element_type=jnp.float32).astype(jnp.bfloat16)
        xwh_ref[...] = jnp.dot(x, wh_ref[...],
                               preferred_element_type=jnp.float32).astype(jnp.bfloat16)
        om_ref[...] = jnp.maximum(
            jnp.dot(x, wm_ref[...], preferred_element_type=jnp.float32), 0.0)

    slot = i % _NBUF
    pltpu.make_async_copy(al_hbm.at[pl.ds(i * bm, bm), :],
                          al_buf.at[slot], sems.at[0, slot]).wait()
    pltpu.make_async_copy(ah_hbm.at[pl.ds(i * bm, bm), :],
                          ah_buf.at[slot], sems.at[1, slot]).wait()
    al = al_buf[slot].astype(jnp.bfloat16)
    ah = ah_buf[slot].astype(jnp.bfloat16)
    ol = jnp.dot(al, xwl_ref[...], preferred_element_type=jnp.float32)
    oh = jnp.dot(ah, xwh_ref[...], preferred_element_type=jnp.float32)
    ol = jnp.maximum(ol, 0.0)
    oh = jnp.maximum(oh, 0.0)
    om = om_ref[pl.ds(i * bm, bm), :]
    fl = jax.nn.sigmoid(jnp.sum(ol * avl_ref[...], axis=1, keepdims=True))
    fh = jax.nn.sigmoid(jnp.sum(oh * avh_ref[...], axis=1, keepdims=True))
    fm = jax.nn.sigmoid(jnp.sum(om * avm_ref[...], axis=1, keepdims=True))
    a3 = a3_ref[...]
    inv_t = 1.0 / 3.0
    l0 = (fl * a3[0, 0] + fh * a3[1, 0] + fm * a3[2, 0]) * inv_t
    l1 = (fl * a3[0, 1] + fh * a3[1, 1] + fm * a3[2, 1]) * inv_t
    l2 = (fl * a3[0, 2] + fh * a3[1, 2] + fm * a3[2, 2]) * inv_t
    m = jnp.maximum(jnp.maximum(l0, l1), l2)
    e0 = jnp.exp(l0 - m)
    e1 = jnp.exp(l1 - m)
    e2 = jnp.exp(l2 - m)
    scale = 3.0 / (e0 + e1 + e2)
    out_ref[...] = scale * (e0 * ol + e1 * oh + e2 * om)


def kernel(input, adj_low, adj_high, adj_low_unnormalized, W_low, W_high, W_mlp,
           att_vec_low, att_vec_high, att_vec_mlp, att_vec_3):
    n, din = input.shape
    dout = W_low.shape[1]

    bm = 80 if n % 80 == 0 else n
    grid = n // bm

    avl = att_vec_low.reshape(1, dout)
    avh = att_vec_high.reshape(1, dout)
    avm = att_vec_mlp.reshape(1, dout)

    hbm = pl.BlockSpec(memory_space=pl.ANY)
    resident = lambda shape: pl.BlockSpec(shape, lambda i: (0, 0))
    row_out = pl.BlockSpec((bm, dout), lambda i: (i, 0))

    out = pl.pallas_call(
        _gcn_kernel,
        grid=(grid,),
        in_specs=[
            hbm, hbm,
            resident((n, din)), resident((din, dout)), resident((din, dout)),
            resident((din, dout)),
            resident((1, dout)), resident((1, dout)), resident((1, dout)),
            resident((3, 3)),
        ],
        out_specs=row_out,
        out_shape=jax.ShapeDtypeStruct((n, dout), jnp.float32),
        scratch_shapes=[
            pltpu.VMEM((_NBUF, bm, n), jnp.float32),
            pltpu.VMEM((_NBUF, bm, n), jnp.float32),
            pltpu.VMEM((n, dout), jnp.bfloat16),
            pltpu.VMEM((n, dout), jnp.bfloat16),
            pltpu.VMEM((n, dout), jnp.float32),
            pltpu.SemaphoreType.DMA((2, _NBUF)),
        ],
    )(adj_low, adj_high, input, W_low, W_high, W_mlp, avl, avh, avm, att_vec_3)
    return out


# speedup vs baseline: 1.0172x; 1.0048x over previous
import jax
import jax.numpy as jnp
from jax.experimental import pallas as pl
from jax.experimental.pallas import tpu as pltpu

_NBUF = 4


def _gcn_kernel(al_hbm, ah_hbm, x_ref, wl_ref, wh_ref, wm_ref,
                avl_ref, avh_ref, avm_ref, a3_ref, out_ref,
                al_buf, ah_buf, xwl_ref, xwh_ref, om_ref, sems):
    i = pl.program_id(0)
    ni = pl.num_programs(0)
    bm = out_ref.shape[0]

    hm = bm // 2

    def _start(step, slot):
        for h in range(2):
            pltpu.make_async_copy(
                al_hbm.at[pl.ds(step * bm + h * hm, hm), :],
                al_buf.at[slot, pl.ds(h * hm, hm), :],
                sems.at[0, slot, h]).start()
            pltpu.make_async_copy(
                ah_hbm.at[pl.ds(step * bm + h * hm, hm), :],
                ah_buf.at[slot, pl.ds(h * hm, hm), :],
                sems.at[1, slot, h]).start()

    @pl.when(i == 0)
    def _():
        for j in range(_NBUF - 1):
            _start(j, j)

    @pl.when(i + _NBUF - 1 < ni)
    def _():
        step = i + _NBUF - 1
        _start(step, step % _NBUF)

    @pl.when(i == 0)
    def _():
        x = x_ref[...]
        xwl_ref[...] = jnp.dot(x, wl_ref[...],
                               preferred_element_type=jnp.float32).astype(jnp.bfloat16)
        xwh_ref[...] = jnp.dot(x, wh_ref[...],
                               preferred_element_type=jnp.float32).astype(jnp.bfloat16)
        om_ref[...] = jnp.maximum(
            jnp.dot(x, wm_ref[...], preferred_element_type=jnp.float32), 0.0)

    slot = i % _NBUF
    for h in range(2):
        pltpu.make_async_copy(al_hbm.at[pl.ds(i * bm + h * hm, hm), :],
                              al_buf.at[slot, pl.ds(h * hm, hm), :],
                              sems.at[0, slot, h]).wait()
        pltpu.make_async_copy(ah_hbm.at[pl.ds(i * bm + h * hm, hm), :],
                              ah_buf.at[slot, pl.ds(h * hm, hm), :],
                              sems.at[1, slot, h]).wait()
    al = al_buf[slot].astype(jnp.bfloat16)
    ah = ah_buf[slot].astype(jnp.bfloat16)
    ol = jnp.dot(al, xwl_ref[...], preferred_element_type=jnp.float32)
    oh = jnp.dot(ah, xwh_ref[...], preferred_element_type=jnp.float32)
    ol = jnp.maximum(ol, 0.0)
    oh = jnp.maximum(oh, 0.0)
    om = om_ref[pl.ds(i * bm, bm), :]
    fl = jax.nn.sigmoid(jnp.sum(ol * avl_ref[...], axis=1, keepdims=True))
    fh = jax.nn.sigmoid(jnp.sum(oh * avh_ref[...], axis=1, keepdims=True))
    fm = jax.nn.sigmoid(jnp.sum(om * avm_ref[...], axis=1, keepdims=True))
    a3 = a3_ref[...]
    inv_t = 1.0 / 3.0
    l0 = (fl * a3[0, 0] + fh * a3[1, 0] + fm * a3[2, 0]) * inv_t
    l1 = (fl * a3[0, 1] + fh * a3[1, 1] + fm * a3[2, 1]) * inv_t
    l2 = (fl * a3[0, 2] + fh * a3[1, 2] + fm * a3[2, 2]) * inv_t
    m = jnp.maximum(jnp.maximum(l0, l1), l2)
    e0 = jnp.exp(l0 - m)
    e1 = jnp.exp(l1 - m)
    e2 = jnp.exp(l2 - m)
    scale = 3.0 / (e0 + e1 + e2)
    out_ref[...] = scale * (e0 * ol + e1 * oh + e2 * om)


def kernel(input, adj_low, adj_high, adj_low_unnormalized, W_low, W_high, W_mlp,
           att_vec_low, att_vec_high, att_vec_mlp, att_vec_3):
    n, din = input.shape
    dout = W_low.shape[1]

    bm = 80 if n % 80 == 0 else n
    grid = n // bm

    avl = att_vec_low.reshape(1, dout)
    avh = att_vec_high.reshape(1, dout)
    avm = att_vec_mlp.reshape(1, dout)

    hbm = pl.BlockSpec(memory_space=pl.ANY)
    resident = lambda shape: pl.BlockSpec(shape, lambda i: (0, 0))
    row_out = pl.BlockSpec((bm, dout), lambda i: (i, 0))

    out = pl.pallas_call(
        _gcn_kernel,
        grid=(grid,),
        in_specs=[
            hbm, hbm,
            resident((n, din)), resident((din, dout)), resident((din, dout)),
            resident((din, dout)),
            resident((1, dout)), resident((1, dout)), resident((1, dout)),
            resident((3, 3)),
        ],
        out_specs=row_out,
        out_shape=jax.ShapeDtypeStruct((n, dout), jnp.float32),
        scratch_shapes=[
            pltpu.VMEM((_NBUF, bm, n), jnp.float32),
            pltpu.VMEM((_NBUF, bm, n), jnp.float32),
            pltpu.VMEM((n, dout), jnp.bfloat16),
            pltpu.VMEM((n, dout), jnp.bfloat16),
            pltpu.VMEM((n, dout), jnp.float32),
            pltpu.SemaphoreType.DMA((2, _NBUF, 2)),
        ],
    )(adj_low, adj_high, input, W_low, W_high, W_mlp, avl, avh, avm, att_vec_3)
    return out


# speedup vs baseline: 1.0184x; 1.0011x over previous
import jax
import jax.numpy as jnp
from jax.experimental import pallas as pl
from jax.experimental.pallas import tpu as pltpu

_NBUF = 4


def _gcn_kernel(al_hbm, ah_hbm, x_ref, wl_ref, wh_ref, wm_ref,
                avl_ref, avh_ref, avm_ref, a3_ref, out_ref,
                al_buf, ah_buf, xwl_ref, xwh_ref, om_ref, sems):
    i = pl.program_id(0)
    ni = pl.num_programs(0)
    bm = out_ref.shape[0]

    def _start(step, slot):
        pltpu.make_async_copy(al_hbm.at[pl.ds(step * bm, bm), :],
                              al_buf.at[slot], sems.at[0, slot]).start()
        pltpu.make_async_copy(ah_hbm.at[pl.ds(step * bm, bm), :],
                              ah_buf.at[slot], sems.at[1, slot]).start()

    @pl.when(i == 0)
    def _():
        for j in range(_NBUF - 1):
            _start(j, j)

    @pl.when(i + _NBUF - 1 < ni)
    def _():
        step = i + _NBUF - 1
        _start(step, step % _NBUF)

    @pl.when(i == 0)
    def _():
        x = x_ref[...]
        xwl_ref[...] = jnp.dot(x, wl_ref[...],
                               preferred_element_type=jnp.float32).astype(jnp.bfloat16)
        xwh_ref[...] = jnp.dot(x, wh_ref[...],
                               preferred_element_type=jnp.float32).astype(jnp.bfloat16)
        om_ref[...] = jnp.maximum(
            jnp.dot(x, wm_ref[...], preferred_element_type=jnp.float32), 0.0)

    slot = i % _NBUF
    pltpu.make_async_copy(al_hbm.at[pl.ds(i * bm, bm), :],
                          al_buf.at[slot], sems.at[0, slot]).wait()
    pltpu.make_async_copy(ah_hbm.at[pl.ds(i * bm, bm), :],
                          ah_buf.at[slot], sems.at[1, slot]).wait()
    al = al_buf[slot].astype(jnp.bfloat16)
    ah = ah_buf[slot].astype(jnp.bfloat16)
    ol = jnp.dot(al, xwl_ref[...], preferred_element_type=jnp.float32)
    oh = jnp.dot(ah, xwh_ref[...], preferred_element_type=jnp.float32)
    ol = jnp.maximum(ol, 0.0)
    oh = jnp.maximum(oh, 0.0)
    om = om_ref[pl.ds(i * bm, bm), :]
    fl = jax.nn.sigmoid(jnp.sum(ol * avl_ref[...], axis=1, keepdims=True))
    fh = jax.nn.sigmoid(jnp.sum(oh * avh_ref[...], axis=1, keepdims=True))
    fm = jax.nn.sigmoid(jnp.sum(om * avm_ref[...], axis=1, keepdims=True))
    a3 = a3_ref[...]
    inv_t = 1.0 / 3.0
    l0 = (fl * a3[0, 0] + fh * a3[1, 0] + fm * a3[2, 0]) * inv_t
    l1 = (fl * a3[0, 1] + fh * a3[1, 1] + fm * a3[2, 1]) * inv_t
    l2 = (fl * a3[0, 2] + fh * a3[1, 2] + fm * a3[2, 2]) * inv_t
    m = jnp.maximum(jnp.maximum(l0, l1), l2)
    e0 = jnp.exp(l0 - m)
    e1 = jnp.exp(l1 - m)
    e2 = jnp.exp(l2 - m)
    scale = 3.0 / (e0 + e1 + e2)
    out_ref[...] = scale * (e0 * ol + e1 * oh + e2 * om)


def kernel(input, adj_low, adj_high, adj_low_unnormalized, W_low, W_high, W_mlp,
           att_vec_low, att_vec_high, att_vec_mlp, att_vec_3):
    n, din = input.shape
    dout = W_low.shape[1]

    bm = 80 if n % 80 == 0 else n
    grid = n // bm

    avl = att_vec_low.reshape(1, dout)
    avh = att_vec_high.reshape(1, dout)
    avm = att_vec_mlp.reshape(1, dout)

    hbm = pl.BlockSpec(memory_space=pl.ANY)
    resident = lambda shape: pl.BlockSpec(shape, lambda i: (0, 0))
    row_out = pl.BlockSpec((bm, dout), lambda i: (i, 0))

    out = pl.pallas_call(
        _gcn_kernel,
        grid=(grid,),
        in_specs=[
            hbm, hbm,
            resident((n, din)), resident((din, dout)), resident((din, dout)),
            resident((din, dout)),
            resident((1, dout)), resident((1, dout)), resident((1, dout)),
            resident((3, 3)),
        ],
        out_specs=row_out,
        out_shape=jax.ShapeDtypeStruct((n, dout), jnp.float32),
        scratch_shapes=[
            pltpu.VMEM((_NBUF, bm, n), jnp.float32),
            pltpu.VMEM((_NBUF, bm, n), jnp.float32),
            pltpu.VMEM((n, dout), jnp.bfloat16),
            pltpu.VMEM((n, dout), jnp.bfloat16),
            pltpu.VMEM((n, dout), jnp.float32),
            pltpu.SemaphoreType.DMA((2, _NBUF)),
        ],
    )(adj_low, adj_high, input, W_low, W_high, W_mlp, avl, avh, avm, att_vec_3)
    return out
